# baseline (speedup 1.0000x reference)
import jax
import jax.numpy as jnp
from jax import lax
from jax.experimental import pallas as pl
from jax.experimental.pallas import tpu as pltpu
from jax.experimental.pallas import tpu_sc as plsc

N = 10000
D = 128
E = 320000
NC = 2
NS = 16
NW = NC * NS
EPT = E // NW
CH = 128
NCH = 80
NPAD = NCH * CH - EPT
AR = N + 16

_mesh = plsc.VectorSubcoreMesh(core_axis_name="c", subcore_axis_name="s")


NP = 10240


def _deg_body(i0, i1, i2, i3, i4, i5, ones_hbm, zeros_hbm, out, ivec, ones_v,
              sem_s, g0, g1, g2, g3, g4, g5):
    c = lax.axis_index("c")
    s = lax.axis_index("s")
    t = c * NS + s
    idxs = [i0, i1, i2, i3, i4, i5]
    degs = [g0, g1, g2, g3, g4, g5]

    @pl.when(s == 0)
    def _zero():
        for j in range(6):
            pltpu.sync_copy(zeros_hbm, degs[j])

    pltpu.sync_copy(ones_hbm, ones_v)
    plsc.subcore_barrier()

    for j in range(6):
        pltpu.sync_copy(idxs[j].at[t], ivec)

        def _scat(i, _):
            pltpu.async_copy(ones_v, degs[j].at[ivec.at[i]], sem_s, add=True)
            return 0

        lax.fori_loop(0, NCH, _scat, 0)

        def _drain(i, _):
            pltpu.make_async_copy(ones_v, degs[j].at[ivec.at[i]],
                                  sem_s).wait()
            return 0

        lax.fori_loop(0, NCH, _drain, 0)

    plsc.subcore_barrier()

    @pl.when(s < 10)
    def _writeout():
        for j in range(6):
            pltpu.sync_copy(degs[j].at[pl.ds(s * 1024, 1024)],
                            out.at[c, j, 0, pl.ds(s * 1024, 1024)])


@jax.jit
def _sc_degrees(i0, i1, i2, i3, i4, i5, ones, zeros):
    return pl.kernel(
        _deg_body,
        out_type=jax.ShapeDtypeStruct((NC, 6, 1, NP), jnp.float32),
        mesh=_mesh,
        scratch_types=[
            pltpu.VMEM((NCH, CH), jnp.int32),
            pltpu.VMEM((CH,), jnp.float32),
            pltpu.SemaphoreType.DMA,
        ] + [pltpu.VMEM_SHARED((NP,), jnp.float32) for _ in range(6)],
    )(i0, i1, i2, i3, i4, i5, ones, zeros)


def _agg_body(h0, h1, h2, s0, dd0, s1, dd1, s2, dd2, out,
              sidx, didx, rows_a, agg):
    c = lax.axis_index("c")
    s = lax.axis_index("s")
    t = c * NS + s
    hs = [h0, h1, h2]
    srcs = [s0, s1, s2]
    dsts = [dd0, dd1, dd2]
    z16 = jnp.zeros((16,), jnp.float32)

    for r in range(3):
        h = hs[r]

        def _zb(i, _):
            rows_a[i // 8, pl.ds((i % 8) * 16, 16)] = z16
            return 0

        lax.fori_loop(0, CH * 8, _zb, 0, unroll=8)

        @pl.when(s < 10)
        def _zacc():
            def _zrow(z, _):
                pltpu.sync_copy(rows_a, agg.at[pl.ds(s * 1000 + z * CH, CH)])
                return 0

            lax.fori_loop(0, 7, _zrow, 0)
            pltpu.sync_copy(rows_a.at[pl.ds(0, 104)],
                            agg.at[pl.ds(s * 1000 + 896, 104)])

        pltpu.sync_copy(srcs[r].at[t], sidx)
        pltpu.sync_copy(dsts[r].at[t], didx)
        plsc.subcore_barrier()

        def _chunk(i, _):
            pltpu.sync_copy(h.at[sidx.at[i]], rows_a)
            pltpu.sync_copy(rows_a, agg.at[didx.at[i]], add=True)
            return 0

        lax.fori_loop(0, NCH, _chunk, 0)

        plsc.subcore_barrier()

        @pl.when(s < 10)
        def _writeout():
            pltpu.sync_copy(agg.at[pl.ds(s * 1000, 1000)],
                            out.at[r, c, pl.ds(s * 1000, 1000)])

        plsc.subcore_barrier()


@jax.jit
def _sc_aggregate(h0, h1, h2, s0, d0, s1, d1, s2, d2):
    return pl.kernel(
        _agg_body,
        out_type=jax.ShapeDtypeStruct((3, NC, N, D), jnp.float32),
        mesh=_mesh,
        scratch_types=[
            pltpu.VMEM((NCH, CH), jnp.int32),
            pltpu.VMEM((NCH, CH), jnp.int32),
            pltpu.VMEM((CH, D), jnp.float32),
            pltpu.VMEM_SHARED((AR, D), jnp.float32),
        ],
    )(h0, h1, h2, s0, d0, s1, d1, s2, d2)


def _h_body(x_ref, deg_ref, h0_ref, h1_ref, h2_ref):
    xb = x_ref[...]
    hrefs = [h0_ref, h1_ref, h2_ref]
    for r in range(3):
        dsrc = deg_ref[:, 2 * r] + deg_ref[:, 6 + 2 * r]
        norm = jnp.where(dsrc > 0.0, lax.rsqrt(jnp.maximum(dsrc, 1e-12)), 0.0)
        hrefs[r][...] = xb * norm[:, None]


@jax.jit
def _tc_scale(x_p, deg_t):
    br = 2048
    return pl.pallas_call(
        _h_body,
        grid=(NP // br,),
        in_specs=[
            pl.BlockSpec((br, D), lambda i: (i, 0)),
            pl.BlockSpec((br, 12), lambda i: (i, 0)),
        ],
        out_specs=[pl.BlockSpec((br, D), lambda i: (i, 0))] * 3,
        out_shape=[jax.ShapeDtypeStruct((NP, D), jnp.float32)] * 3,
    )(x_p, deg_t)


def _comb_body(p_ref, deg_ref, wc_ref, bm_ref, o_ref):
    cols = []
    for r in range(3):
        din = deg_ref[:, 2 * r + 1] + deg_ref[:, 7 + 2 * r]
        norm = jnp.where(din > 0.0, lax.rsqrt(jnp.maximum(din, 1e-12)), 0.0)
        agg = p_ref[r, 0] + p_ref[r, 1]
        cols.append(agg * norm[:, None])
    a = jnp.concatenate(cols, axis=1)
    o_ref[...] = jnp.dot(a, wc_ref[...],
                         preferred_element_type=jnp.float32) + bm_ref[...]


@jax.jit
def _tc_combine(partials, deg_t, wc, bm):
    br = 1000
    return pl.pallas_call(
        _comb_body,
        grid=(N // br,),
        in_specs=[
            pl.BlockSpec((3, NC, br, D), lambda i: (0, 0, i, 0)),
            pl.BlockSpec((br, 12), lambda i: (i, 0)),
            pl.BlockSpec((3 * D, D), lambda i: (0, 0)),
            pl.BlockSpec((1, D), lambda i: (0, 0)),
        ],
        out_specs=pl.BlockSpec((br, D), lambda i: (i, 0)),
        out_shape=jax.ShapeDtypeStruct((N, D), jnp.float32),
    )(partials, deg_t, wc, bm)


def kernel(x, edge_index_r0, edge_index_r1, edge_index_r2,
           W_r0, b_r0, W_r1, b_r1, W_r2, b_r2):
    e0 = edge_index_r0.astype(jnp.int32)
    e1 = edge_index_r1.astype(jnp.int32)
    e2 = edge_index_r2.astype(jnp.int32)
    pad = jnp.full((NW, NPAD), N, jnp.int32)
    chunk = lambda v: jnp.concatenate(
        [v.reshape(NW, EPT), pad], axis=1).reshape(NW, NCH, CH)
    s0, d0 = chunk(e0[0]), chunk(e0[1])
    s1, d1 = chunk(e1[0]), chunk(e1[1])
    s2, d2 = chunk(e2[0]), chunk(e2[1])

    ones_c = jnp.ones((CH,), jnp.float32)
    zeros_p = jnp.zeros((NP,), jnp.float32)
    deg = _sc_degrees(s0, d0, s1, d1, s2, d2, ones_c, zeros_p)
    deg = deg.reshape(NC, 6, NP)
    deg_tp = jnp.concatenate([deg[0].T, deg[1].T], axis=1)

    x_p = jnp.pad(x, ((0, NP - N), (0, 0)))
    h0, h1, h2 = _tc_scale(x_p, deg_tp)
    partials = _sc_aggregate(h0, h1, h2, s0, d0, s1, d1, s2, d2)

    wc = jnp.concatenate([W_r0, W_r1, W_r2], axis=0) * (1.0 / 3.0)
    bm = ((b_r0 + b_r1 + b_r2) * (1.0 / 3.0)).reshape(1, D)
    return _tc_combine(partials, deg_tp[:N], wc, bm)


# speedup vs baseline: 2.0108x; 2.0108x over previous
import jax
import jax.numpy as jnp
from jax import lax
from jax.experimental import pallas as pl
from jax.experimental.pallas import tpu as pltpu
from jax.experimental.pallas import tpu_sc as plsc

N = 10000
D = 128
E = 320000
NC = 2
NS = 16
NW = NC * NS
EPT = E // NW
CH = 80
NCH = EPT // CH

_mesh = plsc.VectorSubcoreMesh(core_axis_name="c", subcore_axis_name="s")


NP = 10240


def _deg_body(i0, i1, i2, i3, i4, i5, ones_hbm, zeros_hbm, out, ivec, ones_v,
              sem_s, g0, g1, g2, g3, g4, g5):
    c = lax.axis_index("c")
    s = lax.axis_index("s")
    t = c * NS + s
    idxs = [i0, i1, i2, i3, i4, i5]
    degs = [g0, g1, g2, g3, g4, g5]

    @pl.when(s == 0)
    def _zero():
        for j in range(6):
            pltpu.sync_copy(zeros_hbm, degs[j])

    pltpu.sync_copy(ones_hbm, ones_v)
    plsc.subcore_barrier()

    for j in range(6):
        pltpu.sync_copy(idxs[j].at[t], ivec)

        def _scat(i, _):
            pltpu.async_copy(ones_v, degs[j].at[ivec.at[i]], sem_s, add=True)
            return 0

        lax.fori_loop(0, NCH, _scat, 0)

        def _drain(i, _):
            pltpu.make_async_copy(ones_v, degs[j].at[ivec.at[i]],
                                  sem_s).wait()
            return 0

        lax.fori_loop(0, NCH, _drain, 0)

    plsc.subcore_barrier()

    @pl.when(s < 10)
    def _writeout():
        for j in range(6):
            pltpu.sync_copy(degs[j].at[pl.ds(s * 1024, 1024)],
                            out.at[c, j, 0, pl.ds(s * 1024, 1024)])


@jax.jit
def _sc_degrees(i0, i1, i2, i3, i4, i5, ones, zeros):
    return pl.kernel(
        _deg_body,
        out_type=jax.ShapeDtypeStruct((NC, 6, 1, NP), jnp.float32),
        mesh=_mesh,
        scratch_types=[
            pltpu.VMEM((NCH, CH), jnp.int32),
            pltpu.VMEM((CH,), jnp.float32),
            pltpu.SemaphoreType.DMA,
        ] + [pltpu.VMEM_SHARED((NP,), jnp.float32) for _ in range(6)],
    )(i0, i1, i2, i3, i4, i5, ones, zeros)


def _agg_body(h0, h1, h2, s0, dd0, s1, dd1, s2, dd2, out,
              sidx, didx, rows_a, agg):
    c = lax.axis_index("c")
    s = lax.axis_index("s")
    t = c * NS + s
    hs = [h0, h1, h2]
    srcs = [s0, s1, s2]
    dsts = [dd0, dd1, dd2]
    z16 = jnp.zeros((16,), jnp.float32)

    for r in range(3):
        h = hs[r]

        def _zb(i, _):
            rows_a[i // 8, pl.ds((i % 8) * 16, 16)] = z16
            return 0

        lax.fori_loop(0, CH * 8, _zb, 0, unroll=8)

        @pl.when(s < 10)
        def _zacc():
            def _zrow(z, _):
                pltpu.sync_copy(rows_a, agg.at[pl.ds(s * 1000 + z * CH, CH)])
                return 0

            lax.fori_loop(0, 12, _zrow, 0)
            pltpu.sync_copy(rows_a.at[pl.ds(0, 40)],
                            agg.at[pl.ds(s * 1000 + 960, 40)])

        pltpu.sync_copy(srcs[r].at[t], sidx)
        pltpu.sync_copy(dsts[r].at[t], didx)
        plsc.subcore_barrier()

        def _chunk(i, _):
            pltpu.sync_copy(h.at[sidx.at[i]], rows_a)
            pltpu.sync_copy(rows_a, agg.at[didx.at[i]], add=True)
            return 0

        lax.fori_loop(0, NCH, _chunk, 0)

        plsc.subcore_barrier()

        @pl.when(s < 10)
        def _writeout():
            pltpu.sync_copy(agg.at[pl.ds(s * 1000, 1000)],
                            out.at[r, c, pl.ds(s * 1000, 1000)])

        plsc.subcore_barrier()


@jax.jit
def _sc_aggregate(h0, h1, h2, s0, d0, s1, d1, s2, d2):
    return pl.kernel(
        _agg_body,
        out_type=jax.ShapeDtypeStruct((3, NC, N, D), jnp.float32),
        mesh=_mesh,
        scratch_types=[
            pltpu.VMEM((NCH, CH), jnp.int32),
            pltpu.VMEM((NCH, CH), jnp.int32),
            pltpu.VMEM((CH, D), jnp.float32),
            pltpu.VMEM_SHARED((N, D), jnp.float32),
        ],
    )(h0, h1, h2, s0, d0, s1, d1, s2, d2)


def _h_body(x_ref, deg_ref, h0_ref, h1_ref, h2_ref):
    xb = x_ref[...]
    hrefs = [h0_ref, h1_ref, h2_ref]
    for r in range(3):
        dsrc = deg_ref[:, 2 * r] + deg_ref[:, 6 + 2 * r]
        norm = jnp.where(dsrc > 0.0, lax.rsqrt(jnp.maximum(dsrc, 1e-12)), 0.0)
        hrefs[r][...] = xb * norm[:, None]


@jax.jit
def _tc_scale(x, deg_t):
    br = 2000
    return pl.pallas_call(
        _h_body,
        grid=(N // br,),
        in_specs=[
            pl.BlockSpec((br, D), lambda i: (i, 0)),
            pl.BlockSpec((br, 12), lambda i: (i, 0)),
        ],
        out_specs=[pl.BlockSpec((br, D), lambda i: (i, 0))] * 3,
        out_shape=[jax.ShapeDtypeStruct((N, D), jnp.float32)] * 3,
    )(x, deg_t)


def _comb_body(p_ref, deg_ref, wc_ref, bm_ref, o_ref):
    cols = []
    for r in range(3):
        din = deg_ref[:, 2 * r + 1] + deg_ref[:, 7 + 2 * r]
        norm = jnp.where(din > 0.0, lax.rsqrt(jnp.maximum(din, 1e-12)), 0.0)
        agg = p_ref[r, 0] + p_ref[r, 1]
        cols.append(agg * norm[:, None])
    a = jnp.concatenate(cols, axis=1)
    o_ref[...] = jnp.dot(a, wc_ref[...],
                         preferred_element_type=jnp.float32) + bm_ref[...]


@jax.jit
def _tc_combine(partials, deg_t, wc, bm):
    br = 1000
    return pl.pallas_call(
        _comb_body,
        grid=(N // br,),
        in_specs=[
            pl.BlockSpec((3, NC, br, D), lambda i: (0, 0, i, 0)),
            pl.BlockSpec((br, 12), lambda i: (i, 0)),
            pl.BlockSpec((3 * D, D), lambda i: (0, 0)),
            pl.BlockSpec((1, D), lambda i: (0, 0)),
        ],
        out_specs=pl.BlockSpec((br, D), lambda i: (i, 0)),
        out_shape=jax.ShapeDtypeStruct((N, D), jnp.float32),
    )(partials, deg_t, wc, bm)


def kernel(x, edge_index_r0, edge_index_r1, edge_index_r2,
           W_r0, b_r0, W_r1, b_r1, W_r2, b_r2):
    e0 = edge_index_r0.astype(jnp.int32)
    e1 = edge_index_r1.astype(jnp.int32)
    e2 = edge_index_r2.astype(jnp.int32)
    chunk = lambda v: v.reshape(NW, NCH, CH)
    s0, d0 = chunk(e0[0]), chunk(e0[1])
    s1, d1 = chunk(e1[0]), chunk(e1[1])
    s2, d2 = chunk(e2[0]), chunk(e2[1])

    ones_c = jnp.ones((CH,), jnp.float32)
    zeros_p = jnp.zeros((NP,), jnp.float32)
    deg = _sc_degrees(s0, d0, s1, d1, s2, d2, ones_c, zeros_p)
    deg = deg.reshape(NC, 6, NP)[:, :, :N]
    deg_t = jnp.concatenate([deg[0].T, deg[1].T], axis=1)

    h0, h1, h2 = _tc_scale(x, deg_t)
    partials = _sc_aggregate(h0, h1, h2, s0, d0, s1, d1, s2, d2)

    wc = jnp.concatenate([W_r0, W_r1, W_r2], axis=0) * (1.0 / 3.0)
    bm = ((b_r0 + b_r1 + b_r2) * (1.0 / 3.0)).reshape(1, D)
    return _tc_combine(partials, deg_t, wc, bm)
